# baseline (speedup 1.0000x reference)
import jax
import jax.numpy as jnp
from jax import lax
from jax.experimental import pallas as pl
from jax.experimental.pallas import tpu as pltpu
from jax.experimental.pallas import tpu_sc as plsc

B = 16384
V = 100000
EMB = 16
CB_SIZE = 32
CB_DIM = 8

_NC = 2
_NS = 16
_NW = _NC * _NS
_BPW = B // _NW
_GPW = _BPW * EMB


def _sc_gather_body(item_t, brand_t, cate_t, idx_i, idx_b, idx_c,
                    out_i, out_b, out_c,
                    iv0, iv1, iv2, gi0, gi1, gi2, gb0, gb1, gb2,
                    s0, s1, s2, s3, s4, s5):
    wid = lax.axis_index("s") * _NC + lax.axis_index("c")
    base = wid * _BPW
    pltpu.sync_copy(idx_i.at[pl.ds(base, _BPW)], iv0)
    pltpu.sync_copy(idx_b.at[pl.ds(base, _BPW)], iv1)
    pltpu.sync_copy(idx_c.at[pl.ds(base, _BPW)], iv2)

    def expand(c, _):
        b16 = c * 16
        v0 = iv0[pl.ds(b16, 16)]
        v1 = iv1[pl.ds(b16, 16)]
        v2 = iv2[pl.ds(b16, 16)]
        for d in range(EMB):
            off = jnp.full((16,), d * V, jnp.int32)
            gi0[pl.ds(d * _BPW + b16, 16)] = v0 + off
            gi1[pl.ds(d * _BPW + b16, 16)] = v1 + off
            gi2[pl.ds(d * _BPW + b16, 16)] = v2 + off
        return 0

    lax.fori_loop(0, _BPW // 16, expand, 0)

    _H = _GPW // 2
    lo = pl.ds(0, _H)
    hi = pl.ds(_H, _H)
    cs = [
        pltpu.async_copy(item_t.at[gi0.at[lo]], gb0.at[lo], s0),
        pltpu.async_copy(item_t.at[gi0.at[hi]], gb0.at[hi], s3),
        pltpu.async_copy(brand_t.at[gi1.at[lo]], gb1.at[lo], s1),
        pltpu.async_copy(brand_t.at[gi1.at[hi]], gb1.at[hi], s4),
        pltpu.async_copy(cate_t.at[gi2.at[lo]], gb2.at[lo], s2),
        pltpu.async_copy(cate_t.at[gi2.at[hi]], gb2.at[hi], s5),
    ]
    for c in cs:
        c.wait()

    def wb(d, do_start):
        g0 = pltpu.make_async_copy(gb0.at[pl.ds(d * _BPW, _BPW)],
                                   out_i.at[d, pl.ds(base, _BPW)], s0)
        g1 = pltpu.make_async_copy(gb1.at[pl.ds(d * _BPW, _BPW)],
                                   out_b.at[d, pl.ds(base, _BPW)], s1)
        g2 = pltpu.make_async_copy(gb2.at[pl.ds(d * _BPW, _BPW)],
                                   out_c.at[d, pl.ds(base, _BPW)], s2)
        if do_start:
            g0.start(); g1.start(); g2.start()
        else:
            g0.wait(); g1.wait(); g2.wait()

    for d in range(EMB):
        wb(d, True)
    for d in range(EMB):
        wb(d, False)


@jax.jit
def _sc_gather(item_t, brand_t, cate_t, idx_i, idx_b, idx_c):
    mesh = plsc.VectorSubcoreMesh(core_axis_name="c", subcore_axis_name="s")
    out = jax.ShapeDtypeStruct((EMB, B), jnp.float32)
    run = pl.kernel(
        _sc_gather_body,
        mesh=mesh,
        compiler_params=pltpu.CompilerParams(use_tc_tiling_on_sc=False),
        out_type=(out, out, out),
        scratch_types=[
            pltpu.VMEM((_BPW,), jnp.int32),
            pltpu.VMEM((_BPW,), jnp.int32),
            pltpu.VMEM((_BPW,), jnp.int32),
            pltpu.VMEM((_GPW,), jnp.int32),
            pltpu.VMEM((_GPW,), jnp.int32),
            pltpu.VMEM((_GPW,), jnp.int32),
            pltpu.VMEM((_GPW,), jnp.float32),
            pltpu.VMEM((_GPW,), jnp.float32),
            pltpu.VMEM((_GPW,), jnp.float32),
            pltpu.SemaphoreType.DMA,
            pltpu.SemaphoreType.DMA,
            pltpu.SemaphoreType.DMA,
            pltpu.SemaphoreType.DMA,
            pltpu.SemaphoreType.DMA,
            pltpu.SemaphoreType.DMA,
        ],
    )
    return run(item_t, brand_t, cate_t, idx_i, idx_b, idx_c)


_BLK = 16384
_NBLK = B // _BLK


def _dense_body(item_ref, brand_ref, cate_ref, count_ref, noise_ref,
                cb_ref,
                me_w1, me_b1, me_w2, me_b2,
                lv_w1, lv_b1, lv_w2, lv_b2,
                mp_w1, mp_b1, mp_w2, mp_b2,
                lp_w1, lp_b1, lp_w2, lp_b2,
                dec_w1, dec_b1, dec_w2, dec_b2,
                head_w, head_b,
                recon_ref, reg_ref, target_ref, vq_ref, perp_ref,
                counts_ref):
    f32 = jnp.float32
    step = pl.program_id(0)
    item = item_ref[...]
    brand = brand_ref[...]
    cate = cate_ref[...]
    noise = noise_ref[...]
    count = count_ref[...]

    def tmat(w, x):
        return lax.dot_general(w[...], x, (((0,), (0,)), ((), ())),
                               preferred_element_type=f32)

    def col(b):
        return b[...].reshape(-1, 1)

    def mlp2(x, w1, b1, w2, b2):
        h = jnp.maximum(tmat(w1, x) + col(b1), 0.0)
        return tmat(w2, h) + col(b2)

    mean = mlp2(item, me_w1, me_b1, me_w2, me_b2)
    log_v = mlp2(item, lv_w1, lv_b1, lv_w2, lv_b2)

    def mlp2_side(w1, b1, w2, b2):
        h = (lax.dot_general(w1[pl.ds(0, EMB), :], brand,
                             (((0,), (0,)), ((), ())),
                             preferred_element_type=f32)
             + lax.dot_general(w1[pl.ds(EMB, EMB), :], cate,
                               (((0,), (0,)), ((), ())),
                               preferred_element_type=f32)
             + col(b1))
        h = jnp.maximum(h, 0.0)
        return tmat(w2, h) + col(b2)

    mean_p = mlp2_side(mp_w1, mp_b1, mp_w2, mp_b2)
    log_v_p = mlp2_side(lp_w1, lp_b1, lp_w2, lp_b2)

    p1 = jnp.sum(jnp.square(mean - mean_p), axis=0)
    p2 = jnp.sum(jnp.square(jnp.exp(log_v * 0.5) - jnp.exp(log_v_p * 0.5)),
                 axis=0)
    reg_part = jnp.reshape(jnp.sum(p1 + p2), (1, 1))

    z = mean + jnp.exp(log_v * 0.5) * noise

    cb = cb_ref[...]
    cb2 = jnp.sum(cb * cb, axis=1).reshape(-1, 1)
    score = cb2 - 2.0 * jnp.dot(cb, z, preferred_element_type=f32)
    m = jnp.min(score, axis=0, keepdims=True)
    iota = lax.broadcasted_iota(jnp.int32, score.shape, 0)
    idx = jnp.min(jnp.where(score == m, iota, CB_SIZE), axis=0,
                  keepdims=True)
    one_hot = (iota == idx).astype(f32)
    z_q = tmat(cb_ref, one_hot)

    vq_part = jnp.reshape(jnp.sum(jnp.square(z_q - z)), (1, 1))
    counts_part = jnp.sum(one_hot, axis=1).reshape(-1, 1)

    h = (tmat(dec_w1.at[pl.ds(0, CB_DIM), :], z)
         + col(dec_w1.at[CB_DIM]) * count + col(dec_b1))
    h = jnp.maximum(h, 0.0)
    pred = tmat(dec_w2, h) + col(dec_b2)

    recon_part = jnp.reshape(jnp.sum(jnp.square(pred - item)), (1, 1))

    logit = tmat(head_w, pred) + col(head_b)
    target_ref[...] = 1.0 / (1.0 + jnp.exp(-logit))

    @pl.when(step == 0)
    def _init():
        reg_ref[...] = reg_part
        vq_ref[...] = vq_part
        recon_ref[...] = recon_part
        counts_ref[...] = counts_part

    @pl.when(step > 0)
    def _acc():
        reg_ref[...] += reg_part
        vq_ref[...] += vq_part
        recon_ref[...] += recon_part
        counts_ref[...] += counts_part

    @pl.when(step == _NBLK - 1)
    def _finalize():
        vq_ref[...] = vq_ref[...] * (1.25 / (B * CB_DIM))
        recon_ref[...] = recon_ref[...] * (1.0 / B)
        probs = counts_ref[...] * (1.0 / B)
        ent = jnp.sum(probs * jnp.log(probs + 1e-10))
        perp_ref[...] = jnp.reshape(jnp.exp(-ent), (1, 1))


@jax.jit
def _dense(item_t, brand_t, cate_t, count_t, noise_t, codebook,
           me_w1, me_b1, me_w2, me_b2,
           lv_w1, lv_b1, lv_w2, lv_b2,
           mp_w1, mp_b1, mp_w2, mp_b2,
           lp_w1, lp_b1, lp_w2, lp_b2,
           dec_w1, dec_b1, dec_w2, dec_b2,
           head_w, head_b):
    scalar = jax.ShapeDtypeStruct((1, 1), jnp.float32)
    out_shape = (scalar, scalar,
                 jax.ShapeDtypeStruct((1, B), jnp.float32),
                 scalar, scalar)
    blk = lambda i: (0, i)
    cst2 = lambda i: (0, 0)
    cst1 = lambda i: (0,)
    w2spec = pl.BlockSpec(index_map=cst2)
    w1spec = pl.BlockSpec(index_map=cst1)
    tr_spec = lambda h: pl.BlockSpec((h, _BLK), blk)
    scal_spec = pl.BlockSpec((1, 1), cst2)
    wspecs = [w2spec, w1spec, w2spec, w1spec,
              w2spec, w1spec, w2spec, w1spec,
              w2spec, w1spec, w2spec, w1spec,
              w2spec, w1spec, w2spec, w1spec,
              w2spec, w1spec, w2spec, w1spec,
              w2spec, w1spec]
    outs = pl.pallas_call(
        _dense_body,
        grid=(_NBLK,),
        in_specs=[tr_spec(EMB), tr_spec(EMB), tr_spec(EMB),
                  tr_spec(1), tr_spec(CB_DIM), w2spec] + wspecs,
        out_specs=(scal_spec, scal_spec, pl.BlockSpec((1, _BLK), blk),
                   scal_spec, scal_spec),
        scratch_shapes=[pltpu.VMEM((CB_SIZE, 1), jnp.float32)],
        out_shape=out_shape,
    )(item_t, brand_t, cate_t, count_t, noise_t, codebook,
      me_w1, me_b1, me_w2, me_b2,
      lv_w1, lv_b1, lv_w2, lv_b2,
      mp_w1, mp_b1, mp_w2, mp_b2,
      lp_w1, lp_b1, lp_w2, lp_b2,
      dec_w1, dec_b1, dec_w2, dec_b2,
      head_w, head_b)
    recon, reg, target, vq, perp = outs
    return (recon[0, 0], reg[0, 0], target.reshape(B, 1),
            vq[0, 0], perp[0, 0])


def kernel(item_id, feat_brand, feat_cate, count, noise, item_emb_table,
           brand_table, cate_table, codebook,
           me_w1, me_b1, me_w2, me_b2,
           lv_w1, lv_b1, lv_w2, lv_b2,
           mp_w1, mp_b1, mp_w2, mp_b2,
           lp_w1, lp_b1, lp_w2, lp_b2,
           dec_w1, dec_b1, dec_w2, dec_b2,
           head_w, head_b):
    item_t, brand_t, cate_t = _sc_gather(
        item_emb_table.T.reshape(-1), brand_table.T.reshape(-1),
        cate_table.T.reshape(-1),
        item_id.astype(jnp.int32), feat_brand.astype(jnp.int32),
        feat_cate.astype(jnp.int32))
    return _dense(item_t, brand_t, cate_t,
                  count.reshape(1, B), noise.T, codebook,
                  me_w1, me_b1, me_w2, me_b2,
                  lv_w1, lv_b1, lv_w2, lv_b2,
                  mp_w1, mp_b1, mp_w2, mp_b2,
                  lp_w1, lp_b1, lp_w2, lp_b2,
                  dec_w1, dec_b1, dec_w2, dec_b2,
                  head_w, head_b)


# speedup vs baseline: 1.1330x; 1.1330x over previous
import jax
import jax.numpy as jnp
from jax import lax
from jax.experimental import pallas as pl
from jax.experimental.pallas import tpu as pltpu
from jax.experimental.pallas import tpu_sc as plsc

B = 16384
V = 100000
EMB = 16
CB_SIZE = 32
CB_DIM = 8

_NC = 2
_NS = 16
_NW = _NC * _NS
_BPW = B // _NW
_GPW = _BPW * EMB


def _sc_gather_body(tab, idx, out, iv, gi, gb, s0, s1):
    wid = lax.axis_index("s") * _NC + lax.axis_index("c")
    base = wid * _BPW
    pltpu.sync_copy(idx.at[pl.ds(base, _BPW)], iv)

    def expand(c, _):
        b16 = c * 16
        v = iv[pl.ds(b16, 16)]
        for d in range(EMB):
            off = jnp.full((16,), d * V, jnp.int32)
            gi[pl.ds(d * _BPW + b16, 16)] = v + off
        return 0

    lax.fori_loop(0, _BPW // 16, expand, 0)

    _H = _GPW // 2
    lo = pl.ds(0, _H)
    hi = pl.ds(_H, _H)
    c0 = pltpu.async_copy(tab.at[gi.at[lo]], gb.at[lo], s0)
    c1 = pltpu.async_copy(tab.at[gi.at[hi]], gb.at[hi], s1)
    c0.wait()
    c1.wait()

    def wb(d, do_start):
        g = pltpu.make_async_copy(gb.at[pl.ds(d * _BPW, _BPW)],
                                  out.at[d, pl.ds(base, _BPW)],
                                  s0 if d % 2 == 0 else s1)
        if do_start:
            g.start()
        else:
            g.wait()

    for d in range(EMB):
        wb(d, True)
    for d in range(EMB):
        wb(d, False)


@jax.jit
def _sc_gather(tab, idx):
    mesh = plsc.VectorSubcoreMesh(core_axis_name="c", subcore_axis_name="s")
    out = jax.ShapeDtypeStruct((EMB, B), jnp.float32)
    run = pl.kernel(
        _sc_gather_body,
        mesh=mesh,
        compiler_params=pltpu.CompilerParams(use_tc_tiling_on_sc=False),
        out_type=out,
        scratch_types=[
            pltpu.VMEM((_BPW,), jnp.int32),
            pltpu.VMEM((_GPW,), jnp.int32),
            pltpu.VMEM((_GPW,), jnp.float32),
            pltpu.SemaphoreType.DMA,
            pltpu.SemaphoreType.DMA,
        ],
    )
    return run(tab, idx)


_BLK = 16384
_NBLK = B // _BLK


def _dense_body(item_ref, brand_ref, cate_ref, count_ref, noise_ref,
                cb_ref,
                me_w1, me_b1, me_w2, me_b2,
                lv_w1, lv_b1, lv_w2, lv_b2,
                mp_w1, mp_b1, mp_w2, mp_b2,
                lp_w1, lp_b1, lp_w2, lp_b2,
                dec_w1, dec_b1, dec_w2, dec_b2,
                head_w, head_b,
                recon_ref, reg_ref, target_ref, vq_ref, perp_ref,
                counts_ref):
    f32 = jnp.float32
    step = pl.program_id(0)
    item = item_ref[...]
    brand = brand_ref[...]
    cate = cate_ref[...]
    noise = noise_ref[...]
    count = count_ref[...]

    def tmat(w, x):
        return lax.dot_general(w[...], x, (((0,), (0,)), ((), ())),
                               preferred_element_type=f32)

    def col(b):
        return b[...].reshape(-1, 1)

    def mlp2(x, w1, b1, w2, b2):
        h = jnp.maximum(tmat(w1, x) + col(b1), 0.0)
        return tmat(w2, h) + col(b2)

    mean = mlp2(item, me_w1, me_b1, me_w2, me_b2)
    log_v = mlp2(item, lv_w1, lv_b1, lv_w2, lv_b2)

    def mlp2_side(w1, b1, w2, b2):
        h = (lax.dot_general(w1[pl.ds(0, EMB), :], brand,
                             (((0,), (0,)), ((), ())),
                             preferred_element_type=f32)
             + lax.dot_general(w1[pl.ds(EMB, EMB), :], cate,
                               (((0,), (0,)), ((), ())),
                               preferred_element_type=f32)
             + col(b1))
        h = jnp.maximum(h, 0.0)
        return tmat(w2, h) + col(b2)

    mean_p = mlp2_side(mp_w1, mp_b1, mp_w2, mp_b2)
    log_v_p = mlp2_side(lp_w1, lp_b1, lp_w2, lp_b2)

    p1 = jnp.sum(jnp.square(mean - mean_p), axis=0)
    p2 = jnp.sum(jnp.square(jnp.exp(log_v * 0.5) - jnp.exp(log_v_p * 0.5)),
                 axis=0)
    reg_part = jnp.reshape(jnp.sum(p1 + p2), (1, 1))

    z = mean + jnp.exp(log_v * 0.5) * noise

    cb = cb_ref[...]
    cb2 = jnp.sum(cb * cb, axis=1).reshape(-1, 1)
    score = cb2 - 2.0 * jnp.dot(cb, z, preferred_element_type=f32)
    m = jnp.min(score, axis=0, keepdims=True)
    iota = lax.broadcasted_iota(jnp.int32, score.shape, 0)
    idx = jnp.min(jnp.where(score == m, iota, CB_SIZE), axis=0,
                  keepdims=True)
    one_hot = (iota == idx).astype(f32)
    z_q = tmat(cb_ref, one_hot)

    vq_part = jnp.reshape(jnp.sum(jnp.square(z_q - z)), (1, 1))
    counts_part = jnp.sum(one_hot, axis=1).reshape(-1, 1)

    h = (tmat(dec_w1.at[pl.ds(0, CB_DIM), :], z)
         + col(dec_w1.at[CB_DIM]) * count + col(dec_b1))
    h = jnp.maximum(h, 0.0)
    pred = tmat(dec_w2, h) + col(dec_b2)

    recon_part = jnp.reshape(jnp.sum(jnp.square(pred - item)), (1, 1))

    logit = tmat(head_w, pred) + col(head_b)
    target_ref[...] = 1.0 / (1.0 + jnp.exp(-logit))

    @pl.when(step == 0)
    def _init():
        reg_ref[...] = reg_part
        vq_ref[...] = vq_part
        recon_ref[...] = recon_part
        counts_ref[...] = counts_part

    @pl.when(step > 0)
    def _acc():
        reg_ref[...] += reg_part
        vq_ref[...] += vq_part
        recon_ref[...] += recon_part
        counts_ref[...] += counts_part

    @pl.when(step == _NBLK - 1)
    def _finalize():
        vq_ref[...] = vq_ref[...] * (1.25 / (B * CB_DIM))
        recon_ref[...] = recon_ref[...] * (1.0 / B)
        probs = counts_ref[...] * (1.0 / B)
        ent = jnp.sum(probs * jnp.log(probs + 1e-10))
        perp_ref[...] = jnp.reshape(jnp.exp(-ent), (1, 1))


@jax.jit
def _dense(item_t, brand_t, cate_t, count_t, noise_t, codebook,
           me_w1, me_b1, me_w2, me_b2,
           lv_w1, lv_b1, lv_w2, lv_b2,
           mp_w1, mp_b1, mp_w2, mp_b2,
           lp_w1, lp_b1, lp_w2, lp_b2,
           dec_w1, dec_b1, dec_w2, dec_b2,
           head_w, head_b):
    scalar = jax.ShapeDtypeStruct((1, 1), jnp.float32)
    out_shape = (scalar, scalar,
                 jax.ShapeDtypeStruct((1, B), jnp.float32),
                 scalar, scalar)
    blk = lambda i: (0, i)
    cst2 = lambda i: (0, 0)
    cst1 = lambda i: (0,)
    w2spec = pl.BlockSpec(index_map=cst2)
    w1spec = pl.BlockSpec(index_map=cst1)
    tr_spec = lambda h: pl.BlockSpec((h, _BLK), blk)
    scal_spec = pl.BlockSpec((1, 1), cst2)
    wspecs = [w2spec, w1spec, w2spec, w1spec,
              w2spec, w1spec, w2spec, w1spec,
              w2spec, w1spec, w2spec, w1spec,
              w2spec, w1spec, w2spec, w1spec,
              w2spec, w1spec, w2spec, w1spec,
              w2spec, w1spec]
    outs = pl.pallas_call(
        _dense_body,
        grid=(_NBLK,),
        in_specs=[tr_spec(EMB), tr_spec(EMB), tr_spec(EMB),
                  tr_spec(1), tr_spec(CB_DIM), w2spec] + wspecs,
        out_specs=(scal_spec, scal_spec, pl.BlockSpec((1, _BLK), blk),
                   scal_spec, scal_spec),
        scratch_shapes=[pltpu.VMEM((CB_SIZE, 1), jnp.float32)],
        out_shape=out_shape,
    )(item_t, brand_t, cate_t, count_t, noise_t, codebook,
      me_w1, me_b1, me_w2, me_b2,
      lv_w1, lv_b1, lv_w2, lv_b2,
      mp_w1, mp_b1, mp_w2, mp_b2,
      lp_w1, lp_b1, lp_w2, lp_b2,
      dec_w1, dec_b1, dec_w2, dec_b2,
      head_w, head_b)
    recon, reg, target, vq, perp = outs
    return (recon[0, 0], reg[0, 0], target.reshape(B, 1),
            vq[0, 0], perp[0, 0])


def kernel(item_id, feat_brand, feat_cate, count, noise, item_emb_table,
           brand_table, cate_table, codebook,
           me_w1, me_b1, me_w2, me_b2,
           lv_w1, lv_b1, lv_w2, lv_b2,
           mp_w1, mp_b1, mp_w2, mp_b2,
           lp_w1, lp_b1, lp_w2, lp_b2,
           dec_w1, dec_b1, dec_w2, dec_b2,
           head_w, head_b):
    item_t = _sc_gather(item_emb_table.T.reshape(-1),
                        item_id.astype(jnp.int32))
    brand_t = _sc_gather(brand_table.T.reshape(-1),
                         feat_brand.astype(jnp.int32))
    cate_t = _sc_gather(cate_table.T.reshape(-1),
                        feat_cate.astype(jnp.int32))
    return _dense(item_t, brand_t, cate_t,
                  count.reshape(1, B), noise.T, codebook,
                  me_w1, me_b1, me_w2, me_b2,
                  lv_w1, lv_b1, lv_w2, lv_b2,
                  mp_w1, mp_b1, mp_w2, mp_b2,
                  lp_w1, lp_b1, lp_w2, lp_b2,
                  dec_w1, dec_b1, dec_w2, dec_b2,
                  head_w, head_b)
